# baseline (speedup 1.0000x reference)
import functools

import jax
import jax.numpy as jnp
from jax.experimental import pallas as pl

_MASK_PROB = 0.15

_XBLK = 2000
_EBLK = 8000


def _masks():
    key = jax.random.key(42)
    kn, ke = jax.random.split(key)
    node_mask = jax.random.uniform(kn, (10000,)) < _MASK_PROB
    edge_mask = jax.random.uniform(ke, (320000,)) < _MASK_PROB
    node_keep = 1.0 - node_mask.astype(jnp.float32)
    edge_keep = 1.0 - edge_mask.astype(jnp.float32)
    return node_keep.reshape(10000, 1), edge_keep.reshape(320000, 1)


def _body(nx_blocks, nm_ref, em_ref, x_ref, e_ref, ox_ref, oe_ref):
    i = pl.program_id(0)

    @pl.when(i < nx_blocks)
    def _():
        ox_ref[...] = x_ref[...] * nm_ref[...]

    @pl.when(i >= nx_blocks)
    def _():
        oe_ref[...] = e_ref[...] * em_ref[...]


def kernel(x, edge_attr):
    n_nodes, dx = x.shape
    n_edges, de = edge_attr.shape
    node_keep, edge_keep = _masks()

    nx_blocks = n_nodes // _XBLK
    ne_blocks = n_edges // _EBLK
    grid = nx_blocks + ne_blocks

    def x_map(i):
        return (jnp.minimum(i, nx_blocks - 1), 0)

    def e_map(i):
        return (jnp.maximum(i - nx_blocks, 0), 0)

    body = functools.partial(_body, nx_blocks)

    ox, oe = pl.pallas_call(
        body,
        grid=(grid,),
        in_specs=[
            pl.BlockSpec((_XBLK, 1), x_map),
            pl.BlockSpec((_EBLK, 1), e_map),
            pl.BlockSpec((_XBLK, dx), x_map),
            pl.BlockSpec((_EBLK, de), e_map),
        ],
        out_specs=[
            pl.BlockSpec((_XBLK, dx), x_map),
            pl.BlockSpec((_EBLK, de), e_map),
        ],
        out_shape=[
            jax.ShapeDtypeStruct((n_nodes, dx), x.dtype),
            jax.ShapeDtypeStruct((n_edges, de), edge_attr.dtype),
        ],
    )(node_keep, edge_keep, x, edge_attr)

    return ox, oe


# speedup vs baseline: 1.0595x; 1.0595x over previous
import functools

import jax
import jax.numpy as jnp
from jax import lax
from jax.experimental import pallas as pl
from jax.experimental.pallas import tpu as pltpu
from jax.experimental.pallas import tpu_sc as plsc

_MASK_PROB = 0.15

_NW = 32
_N_EDGES = 320000
_EPW = _N_EDGES // _NW
_NCHUNK = 14
_PAD = _NCHUNK * 128


def _prep():
    key = jax.random.key(42)
    kn, ke = jax.random.split(key)
    node_mask = jax.random.uniform(kn, (10000,)) < _MASK_PROB
    edge_mask = jax.random.uniform(ke, (_N_EDGES,)) < _MASK_PROB

    node_keep = (1.0 - node_mask.astype(jnp.float32)).reshape(10000, 1)

    rows = edge_mask.reshape(_NW, _EPW)

    def per_worker(row):
        idx = jnp.nonzero(row, size=_PAD, fill_value=0)[0].astype(jnp.int32)
        cnt = jnp.sum(row.astype(jnp.int32))
        return jnp.where(jnp.arange(_PAD, dtype=jnp.int32) < cnt, idx, idx[0])

    idx_local = jax.vmap(per_worker)(rows)
    idx_global = idx_local + (jnp.arange(_NW, dtype=jnp.int32) * _EPW)[:, None]
    return node_keep, idx_global.reshape(_NW, _NCHUNK, 128)


def _x_body(nm_ref, x_ref, ox_ref):
    ox_ref[...] = x_ref[...] * nm_ref[...]


def _mask_x(node_keep, x):
    n, d = x.shape
    blk = 2000
    return pl.pallas_call(
        _x_body,
        grid=(n // blk,),
        in_specs=[
            pl.BlockSpec((blk, 1), lambda i: (i, 0)),
            pl.BlockSpec((blk, d), lambda i: (i, 0)),
        ],
        out_specs=pl.BlockSpec((blk, d), lambda i: (i, 0)),
        out_shape=jax.ShapeDtypeStruct((n, d), x.dtype),
    )(node_keep, x)


_sc_mesh = plsc.VectorSubcoreMesh(core_axis_name="c", subcore_axis_name="s")


@functools.partial(
    pl.kernel,
    out_type=jax.ShapeDtypeStruct((_N_EDGES, 16), jnp.float32),
    mesh=_sc_mesh,
    scratch_types=[
        pltpu.VMEM((_NCHUNK, 128), jnp.int32),
        pltpu.VMEM((128, 16), jnp.float32),
        pltpu.SemaphoreType.DMA,
    ],
    compiler_params=pltpu.CompilerParams(use_tc_tiling_on_sc=False),
)
def _edge_sc(edge_hbm, idx_hbm, zeros_hbm, out_hbm, idx_v, zeros_v, sem):
    wid = lax.axis_index("s") * 2 + lax.axis_index("c")
    base = wid * _EPW
    pltpu.sync_copy(idx_hbm.at[wid], idx_v)
    pltpu.sync_copy(zeros_hbm, zeros_v)
    pltpu.sync_copy(edge_hbm.at[pl.ds(base, _EPW), :],
                    out_hbm.at[pl.ds(base, _EPW), :])
    copies = [
        pltpu.async_copy(zeros_v, out_hbm.at[idx_v.at[j]], sem)
        for j in range(_NCHUNK)
    ]
    for c in copies:
        c.wait()


def kernel(x, edge_attr):
    node_keep, idx_hbm = _prep()
    zeros = jnp.zeros((128, 16), jnp.float32)
    ox = _mask_x(node_keep, x)
    oe = _edge_sc(edge_attr, idx_hbm, zeros)
    return ox, oe


# speedup vs baseline: 2.3771x; 2.2435x over previous
import functools

import jax
import jax.numpy as jnp
from jax import lax
from jax.experimental import pallas as pl

_MASK_PROB = 0.15

_XBLK = 2000
_EBLK = 8000


def _masks():
    key = jax.random.key(42)
    kn, ke = jax.random.split(key)
    node_mask = jax.random.uniform(kn, (10000,)) < _MASK_PROB
    edge_mask = jax.random.uniform(ke, (320000,)) < _MASK_PROB
    node_keep = (1.0 - node_mask.astype(jnp.float32)).reshape(10000, 1)
    edge_keep = (1.0 - edge_mask.astype(jnp.float32)).reshape(40000, 8)
    return node_keep, edge_keep


def _body(nx_blocks, nm_ref, em_ref, x_ref, e_ref, ox_ref, oe_ref):
    i = pl.program_id(0)

    @pl.when(i < nx_blocks)
    def _():
        ox_ref[...] = x_ref[...] * nm_ref[...]

    @pl.when(i >= nx_blocks)
    def _():
        row = lax.broadcasted_iota(jnp.int32, (8, 128), 0)
        lane = lax.broadcasted_iota(jnp.int32, (8, 128), 1)
        expand = (lane // 16 == row).astype(jnp.float32)
        keep = lax.dot(em_ref[...], expand, preferred_element_type=jnp.float32)
        oe_ref[...] = e_ref[...] * keep


def kernel(x, edge_attr):
    n_nodes, dx = x.shape
    n_edges, de = edge_attr.shape
    node_keep, edge_keep = _masks()

    one = lax.optimization_barrier(jnp.float32(1.0))
    e128 = edge_attr.reshape(n_edges * de // 128, 128) * one
    n_erows = e128.shape[0]

    nx_blocks = n_nodes // _XBLK
    ne_blocks = n_erows // _EBLK
    grid = nx_blocks + ne_blocks

    def x_map(i):
        return (jnp.minimum(i, nx_blocks - 1), 0)

    def e_map(i):
        return (jnp.maximum(i - nx_blocks, 0), 0)

    body = functools.partial(_body, nx_blocks)

    ox, oe = pl.pallas_call(
        body,
        grid=(grid,),
        in_specs=[
            pl.BlockSpec((_XBLK, 1), x_map),
            pl.BlockSpec((_EBLK, 8), e_map),
            pl.BlockSpec((_XBLK, dx), x_map),
            pl.BlockSpec((_EBLK, 128), e_map),
        ],
        out_specs=[
            pl.BlockSpec((_XBLK, dx), x_map),
            pl.BlockSpec((_EBLK, 128), e_map),
        ],
        out_shape=[
            jax.ShapeDtypeStruct((n_nodes, dx), x.dtype),
            jax.ShapeDtypeStruct((n_erows, 128), edge_attr.dtype),
        ],
    )(node_keep, edge_keep, x, e128)

    oe = oe.reshape(n_edges, de) * one
    return ox, oe
